# baseline (speedup 1.0000x reference)
import functools

import jax
import jax.numpy as jnp
from jax import lax
from jax.experimental import pallas as pl
from jax.experimental.pallas import tpu as pltpu
from jax.experimental.pallas import tpu_sc as plsc

_NC = 2
_NS = 16
_NW = _NC * _NS

_ROWS, _COLS = 4096, 200
_B = _ROWS * _COLS
_D = 64
_B_PER_W = _B // _NW
_G = 128
_CHUNK_G = 8
_CHUNK = _CHUNK_G * _G
_N_CHUNKS = _B_PER_W // _CHUNK


def _gather_body(table_hbm, idx_hbm, out_hbm, idx_v, rows_v, gsem):
    wid = lax.axis_index("s") * _NC + lax.axis_index("c")
    base = wid * _B_PER_W

    @pl.loop(0, _N_CHUNKS)
    def _chunk(g):
        start = pl.multiple_of(base + g * _CHUNK, _CHUNK)
        row0 = pl.multiple_of(base // _G + g * _CHUNK_G, _CHUNK_G)
        pltpu.sync_copy(idx_hbm.at[pl.ds(row0, _CHUNK_G)], idx_v)
        copies = []
        for j in range(_CHUNK_G):
            copies.append(
                pltpu.async_copy(
                    table_hbm.at[idx_v.at[j]],
                    rows_v.at[pl.ds(j * _G, _G)],
                    gsem,
                )
            )
        for c in copies:
            c.wait()
        pltpu.sync_copy(rows_v, out_hbm.at[pl.ds(start, _CHUNK)])


_mesh = plsc.VectorSubcoreMesh(core_axis_name="c", subcore_axis_name="s")

_gather = pl.kernel(
    _gather_body,
    out_type=jax.ShapeDtypeStruct((_B, _D), jnp.float32),
    mesh=_mesh,
    compiler_params=pltpu.CompilerParams(use_tc_tiling_on_sc=False),
    scratch_types=[
        pltpu.VMEM((_CHUNK_G, _G), jnp.int32),
        pltpu.VMEM((_CHUNK, _D), jnp.float32),
        pltpu.SemaphoreType.DMA,
    ],
)


def kernel(x, table):
    idx = x.reshape(_B // _G, _G).astype(jnp.int32)
    out = _gather(table, idx)
    return out.reshape(_ROWS, _COLS, _D)


# speedup vs baseline: 1.0123x; 1.0123x over previous
import jax
import jax.numpy as jnp
from jax import lax
from jax.experimental import pallas as pl
from jax.experimental.pallas import tpu as pltpu
from jax.experimental.pallas import tpu_sc as plsc

_NC = 2
_NS = 16
_NW = _NC * _NS

_ROWS, _COLS = 4096, 200
_B = _ROWS * _COLS
_D = 64
_B_PER_W = _B // _NW
_G = 128
_CHUNK_G = 4
_CHUNK = _CHUNK_G * _G
_N_CHUNKS = _B_PER_W // _CHUNK
_NBUF = 2


def _gather_body(table_hbm, idx_hbm, out_hbm, idx_v, rows_v,
                 isem0, isem1, gsem0, gsem1, wsem0, wsem1):
    isem = [isem0, isem1]
    gsem = [gsem0, gsem1]
    wsem = [wsem0, wsem1]
    wid = lax.axis_index("s") * _NC + lax.axis_index("c")
    base = wid * _B_PER_W
    row_base = base // _G

    def idx_copy(c, b):
        row0 = pl.multiple_of(row_base + c * _CHUNK_G, _CHUNK_G)
        return pltpu.make_async_copy(
            idx_hbm.at[pl.ds(row0, _CHUNK_G)], idx_v.at[b], isem[b])

    def wb_copy(c, b):
        start = pl.multiple_of(base + c * _CHUNK, _CHUNK)
        return pltpu.make_async_copy(
            rows_v.at[b], out_hbm.at[pl.ds(start, _CHUNK)], wsem[b])

    for b in range(_NBUF):
        idx_copy(b, b).start()

    @pl.loop(0, _N_CHUNKS, step=_NBUF)
    def _super(g):
        for b in range(_NBUF):
            c = g + b
            @pl.when(c >= _NBUF)
            def _():
                wb_copy(c, b).wait()
            idx_copy(c, b).wait()
            gathers = [
                pltpu.async_copy(
                    table_hbm.at[idx_v.at[b].at[j]],
                    rows_v.at[b].at[pl.ds(j * _G, _G)],
                    gsem[b],
                )
                for j in range(_CHUNK_G)
            ]
            for cp in gathers:
                cp.wait()
            @pl.when(c + _NBUF < _N_CHUNKS)
            def _():
                idx_copy(c + _NBUF, b).start()
            wb_copy(c, b).start()

    for b in range(_NBUF):
        wb_copy(_N_CHUNKS - _NBUF + b, b).wait()


_mesh = plsc.VectorSubcoreMesh(core_axis_name="c", subcore_axis_name="s")

_gather = pl.kernel(
    _gather_body,
    out_type=jax.ShapeDtypeStruct((_B, _D), jnp.float32),
    mesh=_mesh,
    compiler_params=pltpu.CompilerParams(use_tc_tiling_on_sc=False),
    scratch_types=[
        pltpu.VMEM((_NBUF, _CHUNK_G, _G), jnp.int32),
        pltpu.VMEM((_NBUF, _CHUNK, _D), jnp.float32),
        pltpu.SemaphoreType.DMA,
        pltpu.SemaphoreType.DMA,
        pltpu.SemaphoreType.DMA,
        pltpu.SemaphoreType.DMA,
        pltpu.SemaphoreType.DMA,
        pltpu.SemaphoreType.DMA,
    ],
)


def kernel(x, table):
    idx = x.reshape(_B // _G, _G).astype(jnp.int32)
    out = _gather(table, idx)
    return out.reshape(_ROWS, _COLS, _D)


# speedup vs baseline: 1.0135x; 1.0012x over previous
import jax
import jax.numpy as jnp
from jax import lax
from jax.experimental import pallas as pl
from jax.experimental.pallas import tpu as pltpu
from jax.experimental.pallas import tpu_sc as plsc

_NC = 2
_NS = 16
_NW = _NC * _NS

_ROWS, _COLS = 4096, 200
_D = 64
_R_PER_W = _ROWS // _NW
_R_C = 4
_N_CHUNKS = _R_PER_W // _R_C
_NBUF = 2
_SPLITS = ((0, 128), (128, 72))


def _gather_body(x_hbm, table_hbm, out_hbm, idx_v, rows_v,
                 isem0, isem1, gsem0, gsem1, wsem0, wsem1):
    isem = [isem0, isem1]
    gsem = [gsem0, gsem1]
    wsem = [wsem0, wsem1]
    wid = lax.axis_index("s") * _NC + lax.axis_index("c")
    base = wid * _R_PER_W

    def idx_copy(c, b):
        row0 = pl.multiple_of(base + c * _R_C, _R_C)
        return pltpu.make_async_copy(
            x_hbm.at[pl.ds(row0, _R_C)], idx_v.at[b], isem[b])

    def wb_copy(c, b):
        row0 = pl.multiple_of(base + c * _R_C, _R_C)
        return pltpu.make_async_copy(
            rows_v.at[b], out_hbm.at[pl.ds(row0, _R_C)], wsem[b])

    for b in range(_NBUF):
        idx_copy(b, b).start()

    @pl.loop(0, _N_CHUNKS, step=_NBUF)
    def _super(g):
        for b in range(_NBUF):
            c = g + b
            @pl.when(c >= _NBUF)
            def _():
                wb_copy(c, b).wait()
            idx_copy(c, b).wait()
            gathers = [
                pltpu.async_copy(
                    table_hbm.at[idx_v.at[b].at[r].at[pl.ds(off, ln)]],
                    rows_v.at[b].at[r].at[pl.ds(off, ln)],
                    gsem[b],
                )
                for r in range(_R_C)
                for off, ln in _SPLITS
            ]
            for cp in gathers:
                cp.wait()
            @pl.when(c + _NBUF < _N_CHUNKS)
            def _():
                idx_copy(c + _NBUF, b).start()
            wb_copy(c, b).start()

    for b in range(_NBUF):
        wb_copy(_N_CHUNKS - _NBUF + b, b).wait()


_mesh = plsc.VectorSubcoreMesh(core_axis_name="c", subcore_axis_name="s")

_gather = pl.kernel(
    _gather_body,
    out_type=jax.ShapeDtypeStruct((_ROWS, _COLS, _D), jnp.float32),
    mesh=_mesh,
    compiler_params=pltpu.CompilerParams(use_tc_tiling_on_sc=False),
    scratch_types=[
        pltpu.VMEM((_NBUF, _R_C, _COLS), jnp.int32),
        pltpu.VMEM((_NBUF, _R_C, _COLS, _D), jnp.float32),
        pltpu.SemaphoreType.DMA,
        pltpu.SemaphoreType.DMA,
        pltpu.SemaphoreType.DMA,
        pltpu.SemaphoreType.DMA,
        pltpu.SemaphoreType.DMA,
        pltpu.SemaphoreType.DMA,
    ],
)


def kernel(x, table):
    return _gather(x.astype(jnp.int32), table)
